# Initial kernel scaffold; baseline (speedup 1.0000x reference)
import jax
import jax.numpy as jnp
from jax.experimental import pallas as pl


def _bcast_kernel(tab_ref, out_ref):
    out_ref[...] = jnp.broadcast_to(tab_ref[None, :, :], out_ref.shape)


def kernel(inputs, pembs_weight):
    batch_size, seqs_len = inputs.shape[:2]
    num_units = pembs_weight.shape[1]
    table = pembs_weight[:seqs_len]

    BB = 128
    grid = (batch_size // BB,)

    return pl.pallas_call(
        _bcast_kernel,
        grid=grid,
        in_specs=[
            pl.BlockSpec((seqs_len, num_units), lambda i: (0, 0)),
        ],
        out_specs=pl.BlockSpec((BB, seqs_len, num_units), lambda i: (i, 0, 0)),
        out_shape=jax.ShapeDtypeStruct(
            (batch_size, seqs_len, num_units), pembs_weight.dtype
        ),
    )(table)


# speedup vs baseline: 22.7741x; 22.7741x over previous
import jax
import jax.numpy as jnp
from jax.experimental import pallas as pl


def _bcast_kernel(tab_ref, out_ref):
    out_ref[...] = jnp.broadcast_to(tab_ref[...][None, :, :], out_ref.shape)


def kernel(inputs, pembs_weight):
    batch_size, seqs_len = inputs.shape[:2]
    num_units = pembs_weight.shape[1]
    table = pembs_weight[:seqs_len]

    BB = 128
    grid = (batch_size // BB,)

    return pl.pallas_call(
        _bcast_kernel,
        grid=grid,
        in_specs=[
            pl.BlockSpec((seqs_len, num_units), lambda i: (0, 0)),
        ],
        out_specs=pl.BlockSpec((BB, seqs_len, num_units), lambda i: (i, 0, 0)),
        out_shape=jax.ShapeDtypeStruct(
            (batch_size, seqs_len, num_units), pembs_weight.dtype
        ),
    )(table)
